# Initial kernel scaffold; baseline (speedup 1.0000x reference)
import jax
import jax.numpy as jnp
from jax.experimental import pallas as pl


def kernel(wh4, wl4, b_h4, w_et4, w_es4, w_eb4, w_ef4, b_e4, ef_mu, ef_dev, h, lap, et, es, ed, eb, sign_key):
    raise NotImplementedError("write your pallas kernel here")


# speedup vs baseline: 1.0238x; 1.0238x over previous
import jax
import jax.numpy as jnp
from jax.experimental import pallas as pl
from jax.experimental.pallas import tpu as pltpu

_PACK = 4


def _node_body(sgn_ref, h4_ref, lap4_ref, wh_ref, wl_ref, b_ref, out_ref):
    acc = jnp.dot(h4_ref[...], wh_ref[...], preferred_element_type=jnp.float32)
    lap_s = lap4_ref[...] * sgn_ref[...]
    acc = acc + jnp.dot(lap_s, wl_ref[...], preferred_element_type=jnp.float32)
    out_ref[...] = acc + b_ref[...]


def _edge_body(et_ref, es_ref, eb_ref, ed_ref,
               wt_ref, ws_ref, wb_ref, wf_ref, b_ref,
               mu_ref, dev_ref, out_ref):
    mu = mu_ref[0]
    dev = dev_ref[0]
    d = ed_ref[...] - mu
    ef = jnp.exp(-(d * d) / dev)
    acc = jnp.dot(et_ref[...], wt_ref[...], preferred_element_type=jnp.float32)
    acc = acc + jnp.dot(es_ref[...], ws_ref[...],
                        preferred_element_type=jnp.float32)
    acc = acc + jnp.dot(eb_ref[...], wb_ref[...],
                        preferred_element_type=jnp.float32)
    acc = acc + jnp.dot(ef, wf_ref[...], preferred_element_type=jnp.float32)
    out_ref[...] = acc + b_ref[...]


def kernel(wh4, wl4, b_h4, w_et4, w_es4, w_eb4, w_ef4, b_e4, ef_mu, ef_dev,
           h, lap, et, es, ed, eb, sign_key):
    hw = b_h4.shape[1]
    H = hw // _PACK
    P = wl4.shape[0] // _PACK
    n, ne = h.shape[0], et.shape[0]
    n4, e4 = n // _PACK, ne // _PACK

    r = jax.random.uniform(jax.random.wrap_key_data(sign_key), (P,),
                           jnp.float32)
    sign = jnp.where(r >= 0.5, 1.0, -1.0).astype(jnp.float32)
    sgn_row = jnp.tile(sign, _PACK).reshape(1, _PACK * P)

    h4 = h.reshape(n4, _PACK * h.shape[1])
    lap4 = lap.reshape(n4, _PACK * P)
    et4 = et.reshape(e4, _PACK * et.shape[1])
    es4 = es.reshape(e4, _PACK * es.shape[1])
    eb4 = eb.reshape(e4, _PACK * eb.shape[1])
    ed4 = ed.reshape(e4, _PACK * ed.shape[1])

    bn_n, bn_e = 2048, 2048
    dh4, dl4 = h4.shape[1], lap4.shape[1]
    dt4, ds4, db4, dd4 = et4.shape[1], es4.shape[1], eb4.shape[1], ed4.shape[1]

    out_h4 = pl.pallas_call(
        _node_body,
        out_shape=jax.ShapeDtypeStruct((n4, hw), jnp.float32),
        grid=(n4 // bn_n,),
        in_specs=[
            pl.BlockSpec((1, dl4), lambda i: (0, 0)),
            pl.BlockSpec((bn_n, dh4), lambda i: (i, 0)),
            pl.BlockSpec((bn_n, dl4), lambda i: (i, 0)),
            pl.BlockSpec((dh4, hw), lambda i: (0, 0)),
            pl.BlockSpec((dl4, hw), lambda i: (0, 0)),
            pl.BlockSpec((1, hw), lambda i: (0, 0)),
        ],
        out_specs=pl.BlockSpec((bn_n, hw), lambda i: (i, 0)),
        compiler_params=pltpu.CompilerParams(dimension_semantics=("parallel",)),
    )(sgn_row, h4, lap4, wh4, wl4, b_h4)

    out_e4 = pl.pallas_call(
        _edge_body,
        out_shape=jax.ShapeDtypeStruct((e4, hw), jnp.float32),
        grid=(e4 // bn_e,),
        in_specs=[
            pl.BlockSpec((bn_e, dt4), lambda i: (i, 0)),
            pl.BlockSpec((bn_e, ds4), lambda i: (i, 0)),
            pl.BlockSpec((bn_e, db4), lambda i: (i, 0)),
            pl.BlockSpec((bn_e, dd4), lambda i: (i, 0)),
            pl.BlockSpec((dt4, hw), lambda i: (0, 0)),
            pl.BlockSpec((ds4, hw), lambda i: (0, 0)),
            pl.BlockSpec((db4, hw), lambda i: (0, 0)),
            pl.BlockSpec((dd4, hw), lambda i: (0, 0)),
            pl.BlockSpec((1, hw), lambda i: (0, 0)),
            pl.BlockSpec(memory_space=pltpu.MemorySpace.SMEM),
            pl.BlockSpec(memory_space=pltpu.MemorySpace.SMEM),
        ],
        out_specs=pl.BlockSpec((bn_e, hw), lambda i: (i, 0)),
        compiler_params=pltpu.CompilerParams(dimension_semantics=("parallel",)),
    )(et4, es4, eb4, ed4, w_et4, w_es4, w_eb4, w_ef4, b_e4, ef_mu, ef_dev)

    h_out = out_h4.reshape(n, H)
    e_out = out_e4.reshape(ne, H)
    return h_out, e_out, sign.reshape(1, -1)


# speedup vs baseline: 26.2486x; 25.6391x over previous
import jax
import jax.numpy as jnp
from jax.experimental import pallas as pl
from jax.experimental.pallas import tpu as pltpu

_PACK = 4


def _node_body(ht_ref, lt_ref, a_ref, l_ref, b_ref, out_ref):
    acc = jnp.dot(a_ref[...], ht_ref[...], preferred_element_type=jnp.float32)
    acc = acc + jnp.dot(l_ref[...], lt_ref[...],
                        preferred_element_type=jnp.float32)
    out_ref[...] = acc + b_ref[...]


def _edge_body(et_ref, es_ref, eb_ref, ed_ref,
               wt_ref, ws_ref, wb_ref, wf_ref, b_ref,
               mu_ref, dev_ref, out_ref):
    mu = mu_ref[0]
    dev = dev_ref[0]
    d = ed_ref[...] - mu
    ef = jnp.exp(-(d * d) / dev)
    acc = jnp.dot(wt_ref[...], et_ref[...], preferred_element_type=jnp.float32)
    acc = acc + jnp.dot(ws_ref[...], es_ref[...],
                        preferred_element_type=jnp.float32)
    acc = acc + jnp.dot(wb_ref[...], eb_ref[...],
                        preferred_element_type=jnp.float32)
    acc = acc + jnp.dot(wf_ref[...], ef, preferred_element_type=jnp.float32)
    out_ref[...] = acc + b_ref[...]


def kernel(wh4, wl4, b_h4, w_et4, w_es4, w_eb4, w_ef4, b_e4, ef_mu, ef_dev,
           h, lap, et, es, ed, eb, sign_key):
    H = b_h4.shape[1] // _PACK
    P = wl4.shape[0] // _PACK
    n, ne = h.shape[0], et.shape[0]
    dh, dt, ds, db, dd = (h.shape[1], et.shape[1], es.shape[1], eb.shape[1],
                          ed.shape[1])

    r = jax.random.uniform(jax.random.wrap_key_data(sign_key), (P,),
                           jnp.float32)
    sign = jnp.where(r >= 0.5, 1.0, -1.0).astype(jnp.float32)

    a_t = wh4[:dh, :H].T
    l_t = wl4[:P, :H].T * sign[None, :]
    b_h = b_h4[:1, :H].T
    wt_t = w_et4[:dt, :H].T
    ws_t = w_es4[:ds, :H].T
    wb_t = w_eb4[:db, :H].T
    wf_t = w_ef4[:dd, :H].T
    b_e = b_e4[:1, :H].T

    ht, lt = h.T, lap.T
    ett, est, ebt, edt = et.T, es.T, eb.T, ed.T

    bw_n, bw_e = 32768, 32768

    out_ht = pl.pallas_call(
        _node_body,
        out_shape=jax.ShapeDtypeStruct((H, n), jnp.float32),
        grid=(pl.cdiv(n, bw_n),),
        in_specs=[
            pl.BlockSpec((dh, bw_n), lambda i: (0, i)),
            pl.BlockSpec((P, bw_n), lambda i: (0, i)),
            pl.BlockSpec((H, dh), lambda i: (0, 0)),
            pl.BlockSpec((H, P), lambda i: (0, 0)),
            pl.BlockSpec((H, 1), lambda i: (0, 0)),
        ],
        out_specs=pl.BlockSpec((H, bw_n), lambda i: (0, i)),
        compiler_params=pltpu.CompilerParams(dimension_semantics=("parallel",)),
    )(ht, lt, a_t, l_t, b_h)

    out_et = pl.pallas_call(
        _edge_body,
        out_shape=jax.ShapeDtypeStruct((H, ne), jnp.float32),
        grid=(pl.cdiv(ne, bw_e),),
        in_specs=[
            pl.BlockSpec((dt, bw_e), lambda i: (0, i)),
            pl.BlockSpec((ds, bw_e), lambda i: (0, i)),
            pl.BlockSpec((db, bw_e), lambda i: (0, i)),
            pl.BlockSpec((dd, bw_e), lambda i: (0, i)),
            pl.BlockSpec((H, dt), lambda i: (0, 0)),
            pl.BlockSpec((H, ds), lambda i: (0, 0)),
            pl.BlockSpec((H, db), lambda i: (0, 0)),
            pl.BlockSpec((H, dd), lambda i: (0, 0)),
            pl.BlockSpec((H, 1), lambda i: (0, 0)),
            pl.BlockSpec(memory_space=pltpu.MemorySpace.SMEM),
            pl.BlockSpec(memory_space=pltpu.MemorySpace.SMEM),
        ],
        out_specs=pl.BlockSpec((H, bw_e), lambda i: (0, i)),
        compiler_params=pltpu.CompilerParams(dimension_semantics=("parallel",)),
    )(ett, est, ebt, edt, wt_t, ws_t, wb_t, wf_t, b_e, ef_mu, ef_dev)

    return out_ht.T, out_et.T, sign.reshape(1, -1)


# speedup vs baseline: 28.8814x; 1.1003x over previous
import jax
import jax.numpy as jnp
from jax.experimental import pallas as pl
from jax.experimental.pallas import tpu as pltpu

_PACK = 4


def _node_body(ht_ref, lt_ref, a_ref, l_ref, b_ref, out_ref):
    acc = jnp.dot(a_ref[...], ht_ref[...], preferred_element_type=jnp.float32)
    acc = acc + jnp.dot(l_ref[...], lt_ref[...],
                        preferred_element_type=jnp.float32)
    out_ref[...] = acc + b_ref[...]


def _edge_body(et_ref, es_ref, eb_ref, ed_ref,
               wt_ref, ws_ref, wb_ref, wf_ref, b_ref,
               mu_ref, dev_ref, out_ref):
    mu = mu_ref[0]
    dev = dev_ref[0]
    d = ed_ref[...] - mu
    ef = jnp.exp(-(d * d) / dev)
    acc = jnp.dot(wt_ref[...], et_ref[...], preferred_element_type=jnp.float32)
    acc = acc + jnp.dot(ws_ref[...], es_ref[...],
                        preferred_element_type=jnp.float32)
    acc = acc + jnp.dot(wb_ref[...], eb_ref[...],
                        preferred_element_type=jnp.float32)
    acc = acc + jnp.dot(wf_ref[...], ef, preferred_element_type=jnp.float32)
    out_ref[...] = acc + b_ref[...]


def kernel(wh4, wl4, b_h4, w_et4, w_es4, w_eb4, w_ef4, b_e4, ef_mu, ef_dev,
           h, lap, et, es, ed, eb, sign_key):
    H = b_h4.shape[1] // _PACK
    P = wl4.shape[0] // _PACK
    n, ne = h.shape[0], et.shape[0]
    dh, dt, ds, db, dd = (h.shape[1], et.shape[1], es.shape[1], eb.shape[1],
                          ed.shape[1])

    r = jax.random.uniform(jax.random.wrap_key_data(sign_key), (P,),
                           jnp.float32)
    sign = jnp.where(r >= 0.5, 1.0, -1.0).astype(jnp.float32)

    a_t = wh4[:dh, :H].T
    l_t = wl4[:P, :H].T * sign[None, :]
    b_h = b_h4[:1, :H].T
    wt_t = w_et4[:dt, :H].T
    ws_t = w_es4[:ds, :H].T
    wb_t = w_eb4[:db, :H].T
    wf_t = w_ef4[:dd, :H].T
    b_e = b_e4[:1, :H].T

    ht, lt = h.T, lap.T
    ett, est, ebt, edt = et.T, es.T, eb.T, ed.T

    bw_n, bw_e = 65536, 65536

    out_ht = pl.pallas_call(
        _node_body,
        out_shape=jax.ShapeDtypeStruct((H, n), jnp.float32),
        grid=(pl.cdiv(n, bw_n),),
        in_specs=[
            pl.BlockSpec((dh, bw_n), lambda i: (0, i)),
            pl.BlockSpec((P, bw_n), lambda i: (0, i)),
            pl.BlockSpec((H, dh), lambda i: (0, 0)),
            pl.BlockSpec((H, P), lambda i: (0, 0)),
            pl.BlockSpec((H, 1), lambda i: (0, 0)),
        ],
        out_specs=pl.BlockSpec((H, bw_n), lambda i: (0, i)),
        compiler_params=pltpu.CompilerParams(dimension_semantics=("parallel",)),
    )(ht, lt, a_t, l_t, b_h)

    out_et = pl.pallas_call(
        _edge_body,
        out_shape=jax.ShapeDtypeStruct((H, ne), jnp.float32),
        grid=(pl.cdiv(ne, bw_e),),
        in_specs=[
            pl.BlockSpec((dt, bw_e), lambda i: (0, i)),
            pl.BlockSpec((ds, bw_e), lambda i: (0, i)),
            pl.BlockSpec((db, bw_e), lambda i: (0, i)),
            pl.BlockSpec((dd, bw_e), lambda i: (0, i)),
            pl.BlockSpec((H, dt), lambda i: (0, 0)),
            pl.BlockSpec((H, ds), lambda i: (0, 0)),
            pl.BlockSpec((H, db), lambda i: (0, 0)),
            pl.BlockSpec((H, dd), lambda i: (0, 0)),
            pl.BlockSpec((H, 1), lambda i: (0, 0)),
            pl.BlockSpec(memory_space=pltpu.MemorySpace.SMEM),
            pl.BlockSpec(memory_space=pltpu.MemorySpace.SMEM),
        ],
        out_specs=pl.BlockSpec((H, bw_e), lambda i: (0, i)),
        compiler_params=pltpu.CompilerParams(dimension_semantics=("parallel",)),
    )(ett, est, ebt, edt, wt_t, ws_t, wb_t, wf_t, b_e, ef_mu, ef_dev)

    return out_ht.T, out_et.T, sign.reshape(1, -1)


# speedup vs baseline: 29.0889x; 1.0072x over previous
import jax
import jax.numpy as jnp
from jax.experimental import pallas as pl
from jax.experimental.pallas import tpu as pltpu

_PACK = 4


def _node_body(ht_ref, lt_ref, a_ref, l_ref, b_ref, out_ref):
    acc = jnp.dot(a_ref[...], ht_ref[...], preferred_element_type=jnp.float32)
    acc = acc + jnp.dot(l_ref[...], lt_ref[...],
                        preferred_element_type=jnp.float32)
    out_ref[...] = acc + b_ref[...]


def _edge_body(et_ref, es_ref, eb_ref, ed_ref,
               wt_ref, ws_ref, wb_ref, wf_ref, b_ref,
               mu_ref, dev_ref, out_ref):
    mu = mu_ref[0]
    dev = dev_ref[0]
    d = ed_ref[...] - mu
    ef = jnp.exp(-(d * d) / dev)
    acc = jnp.dot(wt_ref[...], et_ref[...], preferred_element_type=jnp.float32)
    acc = acc + jnp.dot(ws_ref[...], es_ref[...],
                        preferred_element_type=jnp.float32)
    acc = acc + jnp.dot(wb_ref[...], eb_ref[...],
                        preferred_element_type=jnp.float32)
    acc = acc + jnp.dot(wf_ref[...], ef, preferred_element_type=jnp.float32)
    out_ref[...] = acc + b_ref[...]


def kernel(wh4, wl4, b_h4, w_et4, w_es4, w_eb4, w_ef4, b_e4, ef_mu, ef_dev,
           h, lap, et, es, ed, eb, sign_key):
    H = b_h4.shape[1] // _PACK
    P = wl4.shape[0] // _PACK
    n, ne = h.shape[0], et.shape[0]
    dh, dt, ds, db, dd = (h.shape[1], et.shape[1], es.shape[1], eb.shape[1],
                          ed.shape[1])

    r = jax.random.uniform(jax.random.wrap_key_data(sign_key), (P,),
                           jnp.float32)
    sign = jnp.where(r >= 0.5, 1.0, -1.0).astype(jnp.float32)

    a_t = wh4[:dh, :H].T
    l_t = wl4[:P, :H].T * sign[None, :]
    b_h = b_h4[:1, :H].T
    wt_t = w_et4[:dt, :H].T
    ws_t = w_es4[:ds, :H].T
    wb_t = w_eb4[:db, :H].T
    wf_t = w_ef4[:dd, :H].T
    b_e = b_e4[:1, :H].T

    ht, lt = h.T, lap.T
    ett, est, ebt, edt = et.T, es.T, eb.T, ed.T

    bw_n, bw_e = 131072, 65536

    out_ht = pl.pallas_call(
        _node_body,
        out_shape=jax.ShapeDtypeStruct((H, n), jnp.float32),
        grid=(pl.cdiv(n, bw_n),),
        in_specs=[
            pl.BlockSpec((dh, bw_n), lambda i: (0, i)),
            pl.BlockSpec((P, bw_n), lambda i: (0, i)),
            pl.BlockSpec((H, dh), lambda i: (0, 0)),
            pl.BlockSpec((H, P), lambda i: (0, 0)),
            pl.BlockSpec((H, 1), lambda i: (0, 0)),
        ],
        out_specs=pl.BlockSpec((H, bw_n), lambda i: (0, i)),
        compiler_params=pltpu.CompilerParams(dimension_semantics=("parallel",)),
    )(ht, lt, a_t, l_t, b_h)

    out_et = pl.pallas_call(
        _edge_body,
        out_shape=jax.ShapeDtypeStruct((H, ne), jnp.float32),
        grid=(pl.cdiv(ne, bw_e),),
        in_specs=[
            pl.BlockSpec((dt, bw_e), lambda i: (0, i)),
            pl.BlockSpec((ds, bw_e), lambda i: (0, i)),
            pl.BlockSpec((db, bw_e), lambda i: (0, i)),
            pl.BlockSpec((dd, bw_e), lambda i: (0, i)),
            pl.BlockSpec((H, dt), lambda i: (0, 0)),
            pl.BlockSpec((H, ds), lambda i: (0, 0)),
            pl.BlockSpec((H, db), lambda i: (0, 0)),
            pl.BlockSpec((H, dd), lambda i: (0, 0)),
            pl.BlockSpec((H, 1), lambda i: (0, 0)),
            pl.BlockSpec(memory_space=pltpu.MemorySpace.SMEM),
            pl.BlockSpec(memory_space=pltpu.MemorySpace.SMEM),
        ],
        out_specs=pl.BlockSpec((H, bw_e), lambda i: (0, i)),
        compiler_params=pltpu.CompilerParams(dimension_semantics=("parallel",)),
    )(ett, est, ebt, edt, wt_t, ws_t, wb_t, wf_t, b_e, ef_mu, ef_dev)

    return out_ht.T, out_et.T, sign.reshape(1, -1)


# speedup vs baseline: 33.7977x; 1.1619x over previous
import jax
import jax.numpy as jnp
from jax.experimental import pallas as pl
from jax.experimental.pallas import tpu as pltpu

_PACK = 4


def _node_body(ht_ref, lt_ref, a_ref, l_ref, b_ref, out_ref):
    acc = jnp.dot(a_ref[...], ht_ref[...], preferred_element_type=jnp.float32)
    acc = acc + jnp.dot(l_ref[...], lt_ref[...],
                        preferred_element_type=jnp.float32)
    out_ref[...] = acc + b_ref[...]


def _edge_body(et_ref, es_ref, eb_ref, ed_ref, w_ref, b_ref,
               mu_ref, dev_ref, out_ref, rhs_scr):
    mu = mu_ref[0]
    dev = dev_ref[0]
    d = ed_ref[...] - mu
    ef = jnp.exp(-(d * d) / dev)
    rhs_scr[0:5, :] = et_ref[...]
    rhs_scr[5:6, :] = es_ref[...]
    rhs_scr[6:9, :] = eb_ref[...]
    rhs_scr[9:10, :] = ef
    out_ref[...] = jnp.dot(w_ref[...], rhs_scr[...],
                           preferred_element_type=jnp.float32) + b_ref[...]


def kernel(wh4, wl4, b_h4, w_et4, w_es4, w_eb4, w_ef4, b_e4, ef_mu, ef_dev,
           h, lap, et, es, ed, eb, sign_key):
    H = b_h4.shape[1] // _PACK
    P = wl4.shape[0] // _PACK
    n, ne = h.shape[0], et.shape[0]
    dh, dt, ds, db, dd = (h.shape[1], et.shape[1], es.shape[1], eb.shape[1],
                          ed.shape[1])

    r = jax.random.uniform(jax.random.wrap_key_data(sign_key), (P,),
                           jnp.float32)
    sign = jnp.where(r >= 0.5, 1.0, -1.0).astype(jnp.float32)

    a_t = wh4[:dh, :H].T
    l_t = wl4[:P, :H].T * sign[None, :]
    b_h = b_h4[:1, :H].T
    w_edge = jnp.concatenate(
        [w_et4[:dt, :H], w_es4[:ds, :H], w_eb4[:db, :H], w_ef4[:dd, :H]],
        axis=0).T
    b_e = b_e4[:1, :H].T

    ht, lt = h.T, lap.T
    ett, est, ebt, edt = et.T, es.T, eb.T, ed.T

    bw_n, bw_e = 131072, 65536

    out_ht = pl.pallas_call(
        _node_body,
        out_shape=jax.ShapeDtypeStruct((H, n), jnp.float32),
        grid=(pl.cdiv(n, bw_n),),
        in_specs=[
            pl.BlockSpec((dh, bw_n), lambda i: (0, i)),
            pl.BlockSpec((P, bw_n), lambda i: (0, i)),
            pl.BlockSpec((H, dh), lambda i: (0, 0)),
            pl.BlockSpec((H, P), lambda i: (0, 0)),
            pl.BlockSpec((H, 1), lambda i: (0, 0)),
        ],
        out_specs=pl.BlockSpec((H, bw_n), lambda i: (0, i)),
        compiler_params=pltpu.CompilerParams(dimension_semantics=("parallel",)),
    )(ht, lt, a_t, l_t, b_h)

    out_et = pl.pallas_call(
        _edge_body,
        out_shape=jax.ShapeDtypeStruct((H, ne), jnp.float32),
        grid=(pl.cdiv(ne, bw_e),),
        in_specs=[
            pl.BlockSpec((dt, bw_e), lambda i: (0, i)),
            pl.BlockSpec((ds, bw_e), lambda i: (0, i)),
            pl.BlockSpec((db, bw_e), lambda i: (0, i)),
            pl.BlockSpec((dd, bw_e), lambda i: (0, i)),
            pl.BlockSpec((H, dt + ds + db + dd), lambda i: (0, 0)),
            pl.BlockSpec((H, 1), lambda i: (0, 0)),
            pl.BlockSpec(memory_space=pltpu.MemorySpace.SMEM),
            pl.BlockSpec(memory_space=pltpu.MemorySpace.SMEM),
        ],
        out_specs=pl.BlockSpec((H, bw_e), lambda i: (0, i)),
        scratch_shapes=[pltpu.VMEM((dt + ds + db + dd, bw_e), jnp.float32)],
        compiler_params=pltpu.CompilerParams(dimension_semantics=("parallel",)),
    )(ett, est, ebt, edt, w_edge, b_e, ef_mu, ef_dev)

    return out_ht.T, out_et.T, sign.reshape(1, -1)


# speedup vs baseline: 34.3453x; 1.0162x over previous
import jax
import jax.numpy as jnp
from jax.experimental import pallas as pl
from jax.experimental.pallas import tpu as pltpu

_PACK = 4


def _node_body(ht_ref, lt_ref, w_ref, b_ref, out_ref, rhs_scr):
    rhs_scr[0:8, :] = lt_ref[...]
    rhs_scr[8:12, :] = ht_ref[...]
    out_ref[...] = jnp.dot(w_ref[...], rhs_scr[...],
                           preferred_element_type=jnp.float32) + b_ref[...]


def _edge_body(et_ref, es_ref, eb_ref, ed_ref, w_ref, b_ref,
               mu_ref, dev_ref, out_ref, rhs_scr):
    mu = mu_ref[0]
    dev = dev_ref[0]
    d = ed_ref[...] - mu
    ef = jnp.exp(-(d * d) / dev)
    rhs_scr[0:5, :] = et_ref[...]
    rhs_scr[5:6, :] = es_ref[...]
    rhs_scr[6:9, :] = eb_ref[...]
    rhs_scr[9:10, :] = ef
    out_ref[...] = jnp.dot(w_ref[...], rhs_scr[...],
                           preferred_element_type=jnp.float32) + b_ref[...]


def kernel(wh4, wl4, b_h4, w_et4, w_es4, w_eb4, w_ef4, b_e4, ef_mu, ef_dev,
           h, lap, et, es, ed, eb, sign_key):
    H = b_h4.shape[1] // _PACK
    P = wl4.shape[0] // _PACK
    n, ne = h.shape[0], et.shape[0]
    dh, dt, ds, db, dd = (h.shape[1], et.shape[1], es.shape[1], eb.shape[1],
                          ed.shape[1])

    r = jax.random.uniform(jax.random.wrap_key_data(sign_key), (P,),
                           jnp.float32)
    sign = jnp.where(r >= 0.5, 1.0, -1.0).astype(jnp.float32)

    w_node = jnp.concatenate(
        [wl4[:P, :H] * sign[:, None], wh4[:dh, :H]], axis=0).T
    b_h = b_h4[:1, :H].T
    w_edge = jnp.concatenate(
        [w_et4[:dt, :H], w_es4[:ds, :H], w_eb4[:db, :H], w_ef4[:dd, :H]],
        axis=0).T
    b_e = b_e4[:1, :H].T

    ht, lt = h.T, lap.T
    ett, est, ebt, edt = et.T, es.T, eb.T, ed.T

    bw_n, bw_e = 131072, 65536

    out_ht = pl.pallas_call(
        _node_body,
        out_shape=jax.ShapeDtypeStruct((H, n), jnp.float32),
        grid=(pl.cdiv(n, bw_n),),
        in_specs=[
            pl.BlockSpec((dh, bw_n), lambda i: (0, i)),
            pl.BlockSpec((P, bw_n), lambda i: (0, i)),
            pl.BlockSpec((H, P + dh), lambda i: (0, 0)),
            pl.BlockSpec((H, 1), lambda i: (0, 0)),
        ],
        out_specs=pl.BlockSpec((H, bw_n), lambda i: (0, i)),
        scratch_shapes=[pltpu.VMEM((P + dh, bw_n), jnp.float32)],
        compiler_params=pltpu.CompilerParams(dimension_semantics=("parallel",)),
    )(ht, lt, w_node, b_h)

    out_et = pl.pallas_call(
        _edge_body,
        out_shape=jax.ShapeDtypeStruct((H, ne), jnp.float32),
        grid=(pl.cdiv(ne, bw_e),),
        in_specs=[
            pl.BlockSpec((dt, bw_e), lambda i: (0, i)),
            pl.BlockSpec((ds, bw_e), lambda i: (0, i)),
            pl.BlockSpec((db, bw_e), lambda i: (0, i)),
            pl.BlockSpec((dd, bw_e), lambda i: (0, i)),
            pl.BlockSpec((H, dt + ds + db + dd), lambda i: (0, 0)),
            pl.BlockSpec((H, 1), lambda i: (0, 0)),
            pl.BlockSpec(memory_space=pltpu.MemorySpace.SMEM),
            pl.BlockSpec(memory_space=pltpu.MemorySpace.SMEM),
        ],
        out_specs=pl.BlockSpec((H, bw_e), lambda i: (0, i)),
        scratch_shapes=[pltpu.VMEM((dt + ds + db + dd, bw_e), jnp.float32)],
        compiler_params=pltpu.CompilerParams(dimension_semantics=("parallel",)),
    )(ett, est, ebt, edt, w_edge, b_e, ef_mu, ef_dev)

    return out_ht.T, out_et.T, sign.reshape(1, -1)
